# Initial kernel scaffold; baseline (speedup 1.0000x reference)
import jax
import jax.numpy as jnp
from jax.experimental import pallas as pl


def kernel(pred_logits, gt, mask):
    raise NotImplementedError("write your pallas kernel here")


# speedup vs baseline: 18.5214x; 18.5214x over previous
import jax
import jax.numpy as jnp
from jax.experimental import pallas as pl
from jax.experimental.pallas import tpu as pltpu

_NEG_RATIO = 3.0
_EPS = 1e-06

_R = 2048
_C = 1024
_BR = 128
_GRID = _R // _BR


def _body(x_ref, g_ref, m_ref, out_ref, nl_ref, acc_ref):
    i = pl.program_id(0)

    @pl.when(i == 0)
    def _init():
        acc_ref[0] = 0.0
        acc_ref[1] = 0.0
        acc_ref[2] = 0.0

    x = x_ref[...]
    g = g_ref[...]
    m = m_ref[...]
    loss = jnp.maximum(x, 0.0) - x * g + jnp.log(1.0 + jnp.exp(-jnp.abs(x)))
    pos = g * m
    neg = m - pos
    acc_ref[0] += jnp.sum(pos)
    acc_ref[1] += jnp.sum(neg)
    acc_ref[2] += jnp.sum(loss * pos)
    nl_ref[pl.ds(i * _BR, _BR), :] = loss * neg

    @pl.when(i == _GRID - 1)
    def _finish():
        pos_f = acc_ref[0]
        neg_f = acc_ref[1]
        pos_sum = acc_ref[2]
        neg_i = neg_f.astype(jnp.int32)
        k_i = jnp.minimum(neg_i, (pos_f * _NEG_RATIO).astype(jnp.int32))
        k_f = k_i.astype(jnp.float32)
        v = nl_ref[...]

        def bs_step(j, t):
            t_try = t | (jnp.int32(1) << (30 - j))
            thr = jax.lax.bitcast_convert_type(t_try, jnp.float32)
            cnt = jnp.sum((v >= thr).astype(jnp.float32))
            return jnp.where(cnt >= k_f, t_try, t)

        t_bits = jax.lax.fori_loop(0, 31, bs_step, jnp.int32(0))
        thr = jax.lax.bitcast_convert_type(t_bits, jnp.float32)
        gt_mask = v > thr
        cnt_gt = jnp.sum(gt_mask.astype(jnp.float32))
        sum_gt = jnp.sum(jnp.where(gt_mask, v, 0.0))
        top = sum_gt + (k_f - cnt_gt) * thr
        top = jnp.where(k_i > 0, top, 0.0)
        out_ref[0] = (pos_sum + top) / (pos_f + k_f + _EPS)


def kernel(pred_logits, gt, mask):
    x = pred_logits.reshape(_R, _C)
    g = gt.reshape(_R, _C)
    m = mask.reshape(_R, _C)
    out = pl.pallas_call(
        _body,
        grid=(_GRID,),
        in_specs=[
            pl.BlockSpec((_BR, _C), lambda i: (i, 0)),
            pl.BlockSpec((_BR, _C), lambda i: (i, 0)),
            pl.BlockSpec((_BR, _C), lambda i: (i, 0)),
        ],
        out_specs=pl.BlockSpec(memory_space=pltpu.SMEM),
        out_shape=jax.ShapeDtypeStruct((1,), jnp.float32),
        scratch_shapes=[
            pltpu.VMEM((_R, _C), jnp.float32),
            pltpu.SMEM((4,), jnp.float32),
        ],
        compiler_params=pltpu.CompilerParams(
            dimension_semantics=("arbitrary",),
        ),
    )(x, g, m)
    return out.reshape(())


# speedup vs baseline: 46.0536x; 2.4865x over previous
import jax
import jax.numpy as jnp
from jax.experimental import pallas as pl
from jax.experimental.pallas import tpu as pltpu

_NEG_RATIO = 3.0
_EPS = 1e-06

_R = 2048
_C = 1024
_BR = 128
_GRID = _R // _BR


def _body(x_ref, g_ref, m_ref, out_ref, nl_ref, acc_ref):
    i = pl.program_id(0)

    @pl.when(i == 0)
    def _init():
        acc_ref[0] = 0.0
        acc_ref[1] = 0.0
        acc_ref[2] = 0.0
        acc_ref[3] = 0.0

    x = x_ref[...]
    g = g_ref[...]
    m = m_ref[...]
    loss = jnp.maximum(x, 0.0) - x * g + jnp.log(1.0 + jnp.exp(-jnp.abs(x)))
    pos = g * m
    neg = m - pos
    neg_loss = loss * neg
    acc_ref[0] += jnp.sum(pos)
    acc_ref[1] += jnp.sum(neg)
    acc_ref[2] += jnp.sum(loss * pos)
    acc_ref[3] += jnp.sum(neg_loss)
    nl_ref[pl.ds(i * _BR, _BR), :] = neg_loss

    @pl.when(i == _GRID - 1)
    def _finish():
        pos_f = acc_ref[0]
        neg_f = acc_ref[1]
        pos_sum = acc_ref[2]
        neg_i = neg_f.astype(jnp.int32)
        k_i = jnp.minimum(neg_i, (pos_f * _NEG_RATIO).astype(jnp.int32))
        k_f = k_i.astype(jnp.float32)

        @pl.when(k_i < neg_i)
        def _select():
            v = nl_ref[...]

            def bs_step(j, t):
                t_try = t | (jnp.int32(1) << (30 - j))
                thr = jax.lax.bitcast_convert_type(t_try, jnp.float32)
                cnt = jnp.sum((v >= thr).astype(jnp.float32))
                return jnp.where(cnt >= k_f, t_try, t)

            t_bits = jax.lax.fori_loop(0, 31, bs_step, jnp.int32(0))
            thr = jax.lax.bitcast_convert_type(t_bits, jnp.float32)
            gt_mask = v > thr
            cnt_gt = jnp.sum(gt_mask.astype(jnp.float32))
            sum_gt = jnp.sum(jnp.where(gt_mask, v, 0.0))
            top = sum_gt + (k_f - cnt_gt) * thr
            acc_ref[3] = jnp.where(k_i > 0, top, 0.0)

        out_ref[0] = (pos_sum + acc_ref[3]) / (pos_f + k_f + _EPS)


def kernel(pred_logits, gt, mask):
    x = pred_logits.reshape(_R, _C)
    g = gt.reshape(_R, _C)
    m = mask.reshape(_R, _C)
    out = pl.pallas_call(
        _body,
        grid=(_GRID,),
        in_specs=[
            pl.BlockSpec((_BR, _C), lambda i: (i, 0)),
            pl.BlockSpec((_BR, _C), lambda i: (i, 0)),
            pl.BlockSpec((_BR, _C), lambda i: (i, 0)),
        ],
        out_specs=pl.BlockSpec(memory_space=pltpu.SMEM),
        out_shape=jax.ShapeDtypeStruct((1,), jnp.float32),
        scratch_shapes=[
            pltpu.VMEM((_R, _C), jnp.float32),
            pltpu.SMEM((4,), jnp.float32),
        ],
        compiler_params=pltpu.CompilerParams(
            dimension_semantics=("arbitrary",),
        ),
    )(x, g, m)
    return out.reshape(())
